# Initial kernel scaffold; baseline (speedup 1.0000x reference)
import jax
import jax.numpy as jnp
from jax.experimental import pallas as pl


def kernel(inputs, embedding, posembedding):
    raise NotImplementedError("write your pallas kernel here")


# speedup vs baseline: 1.0608x; 1.0608x over previous
import functools

import jax
import jax.numpy as jnp
from jax import lax
from jax.experimental import pallas as pl
from jax.experimental.pallas import tpu as pltpu
from jax.experimental.pallas import tpu_sc as plsc

L = 200
B = 4096
H = 32
FLAT = L * B
NC, NS = 2, 16
NW = NC * NS
PER_W = FLAT // NW
SUB = 128
CH = 1024
NSUB = CH // SUB
NCH = PER_W // CH

_mesh = plsc.VectorSubcoreMesh(core_axis_name="c", subcore_axis_name="s")


@functools.partial(
    pl.kernel,
    out_type=jax.ShapeDtypeStruct((FLAT, H), jnp.float32),
    mesh=_mesh,
    compiler_params=pltpu.CompilerParams(use_tc_tiling_on_sc=False),
    scratch_types=[
        pltpu.VMEM((NSUB, SUB), jnp.int32),
        pltpu.VMEM((CH, H), jnp.float32),
        pltpu.VMEM((H,), jnp.float32),
        pltpu.SemaphoreType.DMA,
    ],
)
def _embed_kernel(idx_hbm, emb_hbm, pos_hbm, out_hbm, idx_v, rows_v, pos_v, gsem):
    wid = lax.axis_index("s") * NC + lax.axis_index("c")
    base = wid * PER_W

    def chunk_body(c, carry):
        start = pl.multiple_of(base + c * CH, CH)
        pltpu.sync_copy(idx_hbm.at[pl.ds(pl.multiple_of(start // SUB, 8), NSUB)], idx_v)
        copies = [
            pltpu.async_copy(
                emb_hbm.at[idx_v.at[j]], rows_v.at[pl.ds(j * SUB, SUB)], gsem
            )
            for j in range(NSUB)
        ]
        pltpu.sync_copy(
            pos_hbm.at[pl.ds(pl.multiple_of((start // B) * H, 8), H)], pos_v
        )
        for cp in copies:
            cp.wait()
        plo = pos_v[pl.ds(0, 16)]
        phi = pos_v[pl.ds(16, 16)]

        @plsc.parallel_loop(0, CH, 1, unroll=8)
        def _add(j):
            rows_v[j, pl.ds(0, 16)] = rows_v[j, pl.ds(0, 16)] + plo
            rows_v[j, pl.ds(16, 16)] = rows_v[j, pl.ds(16, 16)] + phi

        pltpu.sync_copy(rows_v, out_hbm.at[pl.ds(pl.multiple_of(start, 8), CH)])
        return carry

    lax.fori_loop(0, NCH, chunk_body, 0)


def kernel(inputs, embedding, posembedding):
    idx2d = inputs.reshape(FLAT // SUB, SUB)
    pos_flat = posembedding.reshape(L * H)
    out = _embed_kernel(idx2d, embedding, pos_flat)
    return out.reshape(L, B, H)


# speedup vs baseline: 1.1209x; 1.0566x over previous
import functools

import jax
import jax.numpy as jnp
from jax import lax
from jax.experimental import pallas as pl
from jax.experimental.pallas import tpu as pltpu
from jax.experimental.pallas import tpu_sc as plsc

L = 200
B = 4096
H = 32
FLAT = L * B
NC, NS = 2, 16
NW = NC * NS
PER_W = FLAT // NW
SUB = 128
CH = 512
NSUB = CH // SUB
NCH = PER_W // CH
NBUF = 4

_mesh = plsc.VectorSubcoreMesh(core_axis_name="c", subcore_axis_name="s")


@functools.partial(
    pl.kernel,
    out_type=jax.ShapeDtypeStruct((FLAT, H), jnp.float32),
    mesh=_mesh,
    compiler_params=pltpu.CompilerParams(use_tc_tiling_on_sc=False),
    scratch_types=[
        pltpu.VMEM((PER_W // SUB, SUB), jnp.int32),
        pltpu.VMEM((L, H), jnp.float32),
    ]
    + [pltpu.VMEM((CH, H), jnp.float32) for _ in range(NBUF)]
    + [pltpu.SemaphoreType.DMA for _ in range(2 * NBUF)],
)
def _embed_kernel(idx_hbm, emb_hbm, pos_hbm, out_hbm, idx_v, pos_v, *bufs_sems):
    rows = bufs_sems[:NBUF]
    gsem = bufs_sems[NBUF:2 * NBUF]
    osem = bufs_sems[2 * NBUF:]
    wid = lax.axis_index("s") * NC + lax.axis_index("c")
    base = wid * PER_W

    pltpu.sync_copy(
        idx_hbm.at[pl.ds(pl.multiple_of(wid * (PER_W // SUB), 8), PER_W // SUB)],
        idx_v,
    )
    pltpu.sync_copy(pos_hbm, pos_v)

    def fire_gather(c):
        b = c % NBUF
        return [
            pltpu.async_copy(
                emb_hbm.at[idx_v.at[c * NSUB + j]],
                rows[b].at[pl.ds(j * SUB, SUB)],
                gsem[b],
            )
            for j in range(NSUB)
        ]

    gdescs = {}
    odescs = {}
    for c in range(NBUF - 1):
        gdescs[c] = fire_gather(c)

    for c in range(NCH):
        if c + NBUF - 1 < NCH:
            if c - 1 >= 0:
                odescs[c - 1].wait()
            gdescs[c + NBUF - 1] = fire_gather(c + NBUF - 1)
        for d in gdescs.pop(c):
            d.wait()
        b = c % NBUF
        buf = rows[b]
        l = (base + c * CH) // B
        plo = pos_v[l, pl.ds(0, 16)]
        phi = pos_v[l, pl.ds(16, 16)]

        @plsc.parallel_loop(0, CH, 1, unroll=8)
        def _add(j):
            buf[j, pl.ds(0, 16)] = buf[j, pl.ds(0, 16)] + plo
            buf[j, pl.ds(16, 16)] = buf[j, pl.ds(16, 16)] + phi

        odescs[c] = pltpu.async_copy(
            buf,
            out_hbm.at[pl.ds(pl.multiple_of(base + c * CH, 8), CH)],
            osem[b],
        )

    for c in range(NCH - NBUF, NCH):
        odescs[c].wait()


def kernel(inputs, embedding, posembedding):
    idx2d = inputs.reshape(FLAT // SUB, SUB)
    out = _embed_kernel(idx2d, embedding, posembedding)
    return out.reshape(L, B, H)
